# Initial kernel scaffold; baseline (speedup 1.0000x reference)
import jax
import jax.numpy as jnp
from jax.experimental import pallas as pl


def kernel(x, edge_index, x_face, W1, b1, gamma, beta, prelu_a, W2, b2, w0_w, w0_b, gcn_w, gcn_b, wb_w, wb_b):
    raise NotImplementedError("write your pallas kernel here")


# speedup vs baseline: 56.1163x; 56.1163x over previous
import functools

import jax
import jax.numpy as jnp
from jax import lax
from jax.experimental import pallas as pl
from jax.experimental.pallas import tpu as pltpu
from jax.experimental.pallas import tpu_sc as plsc

N = 100000
E = 1600000
H = 32

LANE = 128
N_PAD = 100352
RN = N_PAD // LANE
BLK = 8
NB = RN // BLK
BN = BLK * LANE

NC = 2
NS = 16
NW = NC * NS
SLICE = N_PAD // NS

E_PAD = 1638400
E_ROWS = E_PAD // LANE
ROWS_PER_TILE = E_ROWS // NW
DMA_ROWS = 16
N_CHUNKS = ROWS_PER_TILE // DMA_ROWS


def _moments_body(x_ref, out_ref, acc_ref):
    i = pl.program_id(0)

    @pl.when(i == 0)
    def _init():
        for k in range(5):
            acc_ref[k] = 0.0

    xb = x_ref[...]
    rid = lax.broadcasted_iota(jnp.int32, (BN, 2), 0) + i * BN
    xb = jnp.where(rid < N, xb, 0.0)
    c0 = xb[:, 0]
    c1 = xb[:, 1]
    acc_ref[0] += jnp.sum(c0)
    acc_ref[1] += jnp.sum(c1)
    acc_ref[2] += jnp.sum(c0 * c0)
    acc_ref[3] += jnp.sum(c1 * c1)
    acc_ref[4] += jnp.sum(c0 * c1)

    @pl.when(i == NB - 1)
    def _fin():
        for k in range(5):
            out_ref[k] = acc_ref[k]


def _moments(xp):
    return pl.pallas_call(
        _moments_body,
        grid=(NB,),
        in_specs=[pl.BlockSpec((BN, 2), lambda i: (i, 0))],
        out_specs=pl.BlockSpec(memory_space=pltpu.SMEM),
        out_shape=jax.ShapeDtypeStruct((8,), jnp.float32),
        scratch_shapes=[pltpu.SMEM((8,), jnp.float32)],
    )(xp)


def _node_body(x_ref, mom_ref, w1_ref, gam_ref, bet_ref, vmat_ref, sc_ref,
               s_ref, sc0_ref):
    i = pl.program_id(0)
    inv_n = 1.0 / N
    m0 = mom_ref[0] * inv_n
    m1 = mom_ref[1] * inv_n
    v00 = mom_ref[2] * inv_n - m0 * m0
    v11 = mom_ref[3] * inv_n - m1 * m1
    v01 = mom_ref[4] * inv_n - m0 * m1

    w1 = w1_ref[...]
    w1a = w1[0:1, :]
    w1b = w1[1:2, :]
    varh = v00 * w1a * w1a + 2.0 * v01 * w1a * w1b + v11 * w1b * w1b
    a = gam_ref[...] * lax.rsqrt(varh + 1e-5)
    muc = m0 * w1a + m1 * w1b

    xb = x_ref[...]
    hc = xb[:, 0:1] * w1a + xb[:, 1:2] * w1b - muc
    hn = hc * a + bet_ref[...]
    pa = sc_ref[2]
    act = jnp.where(hn > 0, hn, pa * hn)

    out2 = jnp.dot(act, vmat_ref[...], preferred_element_type=jnp.float32,
                   precision=lax.Precision.HIGHEST)
    sb = out2[:, 0] + sc_ref[0]
    sc0b = out2[:, 1] + sc_ref[1]
    ids = lax.broadcasted_iota(jnp.int32, (BN,), 0) + i * BN
    sb = jnp.where(ids < N, sb, 0.0)
    s_ref[...] = sb.reshape(BLK, LANE)
    sc0_ref[...] = sc0b.reshape(BLK, LANE)


def _node(xp, mom, w1, gam, bet, vmat, scal):
    return pl.pallas_call(
        _node_body,
        grid=(NB,),
        in_specs=[
            pl.BlockSpec((BN, 2), lambda i: (i, 0)),
            pl.BlockSpec(memory_space=pltpu.SMEM),
            pl.BlockSpec((2, H), lambda i: (0, 0)),
            pl.BlockSpec((1, H), lambda i: (0, 0)),
            pl.BlockSpec((1, H), lambda i: (0, 0)),
            pl.BlockSpec((H, 2), lambda i: (0, 0)),
            pl.BlockSpec(memory_space=pltpu.SMEM),
        ],
        out_specs=[
            pl.BlockSpec((BLK, LANE), lambda i: (i, 0)),
            pl.BlockSpec((BLK, LANE), lambda i: (i, 0)),
        ],
        out_shape=[
            jax.ShapeDtypeStruct((RN, LANE), jnp.float32),
            jax.ShapeDtypeStruct((RN, LANE), jnp.float32),
        ],
    )(xp, mom, w1, gam, bet, vmat, scal)


def _hist_body(rows_hbm, zeros_hbm, out_hbm, idx_v, ones_v, deg_sh):
    cid = lax.axis_index("c")
    sid = lax.axis_index("s")
    wid = cid * NS + sid
    for k in range(LANE // 16):
        ones_v[pl.ds(k * 16, 16)] = jnp.ones((16,), jnp.float32)
    sl = pl.ds(sid * SLICE, SLICE)
    pltpu.sync_copy(zeros_hbm.at[sl], deg_sh.at[sl])
    plsc.subcore_barrier()

    base = wid * ROWS_PER_TILE

    def chunk(d, carry):
        pltpu.sync_copy(rows_hbm.at[pl.ds(base + d * DMA_ROWS, DMA_ROWS)],
                        idx_v)
        for j in range(DMA_ROWS):
            pltpu.sync_copy(ones_v, deg_sh.at[idx_v.at[j]], add=True)
        return carry

    lax.fori_loop(0, N_CHUNKS, chunk, 0)
    plsc.subcore_barrier()
    pltpu.sync_copy(deg_sh.at[sl], out_hbm.at[cid, sl])


_hist = functools.partial(
    pl.kernel,
    out_type=jax.ShapeDtypeStruct((NC, N_PAD), jnp.float32),
    mesh=plsc.VectorSubcoreMesh(core_axis_name="c", subcore_axis_name="s", num_cores=NC, num_subcores=NS),
    scratch_types=[
        pltpu.VMEM((DMA_ROWS, LANE), jnp.int32),
        pltpu.VMEM((LANE,), jnp.float32),
        pltpu.VMEM_SHARED((N_PAD,), jnp.float32),
    ],
)(_hist_body)


def _dinv_body(d0_ref, d1_ref, s_ref, t_ref, dinv_ref):
    deg = d0_ref[...] + d1_ref[...] + 1.0
    dinv = lax.rsqrt(deg)
    dinv_ref[...] = dinv
    t_ref[...] = dinv * s_ref[...]


def _dinv_t(d0, d1, s2d):
    return pl.pallas_call(
        _dinv_body,
        grid=(NB,),
        in_specs=[pl.BlockSpec((BLK, LANE), lambda i: (i, 0))] * 3,
        out_specs=[pl.BlockSpec((BLK, LANE), lambda i: (i, 0))] * 2,
        out_shape=[
            jax.ShapeDtypeStruct((RN, LANE), jnp.float32),
            jax.ShapeDtypeStruct((RN, LANE), jnp.float32),
        ],
    )(d0, d1, s2d)


def _edge_body(rows_hbm, cols_hbm, t_hbm, zeros_hbm, out_hbm,
               ridx_v, cidx_v, vals_v, t_sh, g_sh):
    cid = lax.axis_index("c")
    sid = lax.axis_index("s")
    wid = cid * NS + sid
    sl = pl.ds(sid * SLICE, SLICE)
    pltpu.sync_copy(t_hbm.at[sl], t_sh.at[sl])
    pltpu.sync_copy(zeros_hbm.at[sl], g_sh.at[sl])
    plsc.subcore_barrier()

    base = wid * ROWS_PER_TILE

    def chunk(d, carry):
        src = pl.ds(base + d * DMA_ROWS, DMA_ROWS)
        pltpu.sync_copy(rows_hbm.at[src], ridx_v)
        pltpu.sync_copy(cols_hbm.at[src], cidx_v)
        for j in range(DMA_ROWS):
            pltpu.sync_copy(t_sh.at[cidx_v.at[j]], vals_v)
            pltpu.sync_copy(vals_v, g_sh.at[ridx_v.at[j]], add=True)
        return carry

    lax.fori_loop(0, N_CHUNKS, chunk, 0)
    plsc.subcore_barrier()
    pltpu.sync_copy(g_sh.at[sl], out_hbm.at[cid, sl])


_edge = functools.partial(
    pl.kernel,
    out_type=jax.ShapeDtypeStruct((NC, N_PAD), jnp.float32),
    mesh=plsc.VectorSubcoreMesh(core_axis_name="c", subcore_axis_name="s", num_cores=NC, num_subcores=NS),
    scratch_types=[
        pltpu.VMEM((DMA_ROWS, LANE), jnp.int32),
        pltpu.VMEM((DMA_ROWS, LANE), jnp.int32),
        pltpu.VMEM((LANE,), jnp.float32),
        pltpu.VMEM_SHARED((N_PAD,), jnp.float32),
        pltpu.VMEM_SHARED((N_PAD,), jnp.float32),
    ],
)(_edge_body)


def _final_body(sc0_ref, dinv_ref, t_ref, g0_ref, g1_ref, c_ref, out_ref):
    out_ref[...] = sc0_ref[...] + c_ref[0] + dinv_ref[...] * (
        g0_ref[...] + g1_ref[...] + t_ref[...])


def _final(sc02d, dinv2d, t2d, g0, g1, cconst):
    return pl.pallas_call(
        _final_body,
        grid=(NB,),
        in_specs=[pl.BlockSpec((BLK, LANE), lambda i: (i, 0))] * 5
        + [pl.BlockSpec(memory_space=pltpu.SMEM)],
        out_specs=pl.BlockSpec((BLK, LANE), lambda i: (i, 0)),
        out_shape=jax.ShapeDtypeStruct((RN, LANE), jnp.float32),
    )(sc02d, dinv2d, t2d, g0, g1, cconst)


def kernel(x, edge_index, x_face, W1, b1, gamma, beta, prelu_a, W2, b2,
           w0_w, w0_b, gcn_w, gcn_b, wb_w, wb_b):
    f32 = jnp.float32
    row = edge_index[0].astype(jnp.int32)
    col = edge_index[1].astype(jnp.int32)
    pad = jnp.full((E_PAD - E,), N, jnp.int32)
    rows2d = jnp.concatenate([row, pad]).reshape(E_ROWS, LANE)
    cols2d = jnp.concatenate([col, pad]).reshape(E_ROWS, LANE)

    xp = jnp.zeros((N_PAD, 2), f32).at[:N].set(x)
    zeros_n = jnp.zeros((N_PAD,), f32)

    gw = gcn_w @ wb_w
    vmat = jnp.concatenate([W2 @ gw, W2 @ w0_w], axis=1)
    cb = (b2 @ gw)[0]
    c0 = (b2 @ w0_w)[0] + w0_b[0]
    cconst = ((gcn_b @ wb_w)[0] + wb_b[0]).reshape(1)
    scal = jnp.stack([cb, c0, prelu_a[0]])

    mom = _moments(xp)
    s2d, sc02d = _node(xp, mom, W1, gamma.reshape(1, H), beta.reshape(1, H),
                       vmat, scal)

    degp = _hist(rows2d, zeros_n)
    d0 = degp[0].reshape(RN, LANE)
    d1 = degp[1].reshape(RN, LANE)
    t2d, dinv2d = _dinv_t(d0, d1, s2d)

    gp = _edge(rows2d, cols2d, t2d.reshape(N_PAD), zeros_n)
    g0 = gp[0].reshape(RN, LANE)
    g1 = gp[1].reshape(RN, LANE)

    out2d = _final(sc02d, dinv2d, t2d, g0, g1, cconst)
    return out2d.reshape(N_PAD)[:N]


# speedup vs baseline: 60.3437x; 1.0753x over previous
import functools

import jax
import jax.numpy as jnp
from jax import lax
from jax.experimental import pallas as pl
from jax.experimental.pallas import tpu as pltpu
from jax.experimental.pallas import tpu_sc as plsc

N = 100000
E = 1600000
H = 32

LANE = 128
N_PAD = 100352
RN = N_PAD // LANE
BLK = 8
NB = RN // BLK
BN = BLK * LANE

NC = 2
NS = 16
NW = NC * NS
SLICE = N_PAD // NS

E_PAD = 1638400
EPT = E_PAD // NW
ECH = EPT // 4


def _moments_body(x_ref, out_ref, acc_ref):
    i = pl.program_id(0)

    @pl.when(i == 0)
    def _init():
        for k in range(5):
            acc_ref[k] = 0.0

    xb = x_ref[...]
    rid = lax.broadcasted_iota(jnp.int32, (BN, 2), 0) + i * BN
    xb = jnp.where(rid < N, xb, 0.0)
    c0 = xb[:, 0]
    c1 = xb[:, 1]
    acc_ref[0] += jnp.sum(c0)
    acc_ref[1] += jnp.sum(c1)
    acc_ref[2] += jnp.sum(c0 * c0)
    acc_ref[3] += jnp.sum(c1 * c1)
    acc_ref[4] += jnp.sum(c0 * c1)

    @pl.when(i == NB - 1)
    def _fin():
        for k in range(5):
            out_ref[k] = acc_ref[k]


def _moments(xp):
    return pl.pallas_call(
        _moments_body,
        grid=(NB,),
        in_specs=[pl.BlockSpec((BN, 2), lambda i: (i, 0))],
        out_specs=pl.BlockSpec(memory_space=pltpu.SMEM),
        out_shape=jax.ShapeDtypeStruct((8,), jnp.float32),
        scratch_shapes=[pltpu.SMEM((8,), jnp.float32)],
    )(xp)


def _node_body(x_ref, mom_ref, w1_ref, gam_ref, bet_ref, vmat_ref, sc_ref,
               s_ref, sc0_ref):
    i = pl.program_id(0)
    inv_n = 1.0 / N
    m0 = mom_ref[0] * inv_n
    m1 = mom_ref[1] * inv_n
    v00 = mom_ref[2] * inv_n - m0 * m0
    v11 = mom_ref[3] * inv_n - m1 * m1
    v01 = mom_ref[4] * inv_n - m0 * m1

    w1 = w1_ref[...]
    w1a = w1[0:1, :]
    w1b = w1[1:2, :]
    varh = v00 * w1a * w1a + 2.0 * v01 * w1a * w1b + v11 * w1b * w1b
    a = gam_ref[...] * lax.rsqrt(varh + 1e-5)
    muc = m0 * w1a + m1 * w1b

    xb = x_ref[...]
    hc = xb[:, 0:1] * w1a + xb[:, 1:2] * w1b - muc
    hn = hc * a + bet_ref[...]
    pa = sc_ref[2]
    act = jnp.where(hn > 0, hn, pa * hn)

    out2 = jnp.dot(act, vmat_ref[...], preferred_element_type=jnp.float32,
                   precision=lax.Precision.HIGHEST)
    sb = out2[:, 0] + sc_ref[0]
    sc0b = out2[:, 1] + sc_ref[1]
    ids = lax.broadcasted_iota(jnp.int32, (BN,), 0) + i * BN
    sb = jnp.where(ids < N, sb, 0.0)
    s_ref[...] = sb.reshape(BLK, LANE)
    sc0_ref[...] = sc0b.reshape(BLK, LANE)


def _node(xp, mom, w1, gam, bet, vmat, scal):
    return pl.pallas_call(
        _node_body,
        grid=(NB,),
        in_specs=[
            pl.BlockSpec((BN, 2), lambda i: (i, 0)),
            pl.BlockSpec(memory_space=pltpu.SMEM),
            pl.BlockSpec((2, H), lambda i: (0, 0)),
            pl.BlockSpec((1, H), lambda i: (0, 0)),
            pl.BlockSpec((1, H), lambda i: (0, 0)),
            pl.BlockSpec((H, 2), lambda i: (0, 0)),
            pl.BlockSpec(memory_space=pltpu.SMEM),
        ],
        out_specs=[
            pl.BlockSpec((BLK, LANE), lambda i: (i, 0)),
            pl.BlockSpec((BLK, LANE), lambda i: (i, 0)),
        ],
        out_shape=[
            jax.ShapeDtypeStruct((RN, LANE), jnp.float32),
            jax.ShapeDtypeStruct((RN, LANE), jnp.float32),
        ],
    )(xp, mom, w1, gam, bet, vmat, scal)


def _hist_body(rows_hbm, zeros_hbm, ones_hbm, out_hbm, idx_v, ones_v, deg_sh):
    cid = lax.axis_index("c")
    sid = lax.axis_index("s")
    wid = cid * NS + sid
    sl = pl.ds(sid * SLICE, SLICE)
    pltpu.sync_copy(zeros_hbm.at[sl], deg_sh.at[sl])
    pltpu.sync_copy(ones_hbm, ones_v)
    pltpu.sync_copy(rows_hbm.at[pl.ds(wid * EPT, EPT)], idx_v)
    plsc.subcore_barrier()
    pltpu.sync_copy(ones_v, deg_sh.at[idx_v], add=True)
    plsc.subcore_barrier()
    pltpu.sync_copy(deg_sh.at[sl], out_hbm.at[cid, sl])


_hist = functools.partial(
    pl.kernel,
    out_type=jax.ShapeDtypeStruct((NC, N_PAD), jnp.float32),
    mesh=plsc.VectorSubcoreMesh(core_axis_name="c", subcore_axis_name="s", num_cores=NC, num_subcores=NS),
    scratch_types=[
        pltpu.VMEM((EPT,), jnp.int32),
        pltpu.VMEM((EPT,), jnp.float32),
        pltpu.VMEM_SHARED((N_PAD,), jnp.float32),
    ],
)(_hist_body)


def _dinv_body(d0_ref, d1_ref, s_ref, t_ref, dinv_ref):
    deg = d0_ref[...] + d1_ref[...] + 1.0
    dinv = lax.rsqrt(deg)
    dinv_ref[...] = dinv
    t_ref[...] = dinv * s_ref[...]


def _dinv_t(d0, d1, s2d):
    return pl.pallas_call(
        _dinv_body,
        grid=(NB,),
        in_specs=[pl.BlockSpec((BLK, LANE), lambda i: (i, 0))] * 3,
        out_specs=[pl.BlockSpec((BLK, LANE), lambda i: (i, 0))] * 2,
        out_shape=[
            jax.ShapeDtypeStruct((RN, LANE), jnp.float32),
            jax.ShapeDtypeStruct((RN, LANE), jnp.float32),
        ],
    )(d0, d1, s2d)


def _edge_body(rows_hbm, cols_hbm, t_hbm, zeros_hbm, out_hbm,
               ridx_0, ridx_1, ridx_2, ridx_3, cidx_v, vals_v, t_sh, g_sh):
    ridx_bufs = (ridx_0, ridx_1, ridx_2, ridx_3)
    cid = lax.axis_index("c")
    sid = lax.axis_index("s")
    wid = cid * NS + sid
    sl = pl.ds(sid * SLICE, SLICE)
    pltpu.sync_copy(t_hbm.at[sl], t_sh.at[sl])
    pltpu.sync_copy(zeros_hbm.at[sl], g_sh.at[sl])
    base = wid * EPT
    pltpu.sync_copy(cols_hbm.at[pl.ds(base, EPT)], cidx_v)
    for ch, ridx in enumerate(ridx_bufs):
        pltpu.sync_copy(rows_hbm.at[pl.ds(base + ch * ECH, ECH)], ridx)
    plsc.subcore_barrier()
    for ch, ridx in enumerate(ridx_bufs):
        pltpu.sync_copy(t_sh.at[cidx_v.at[pl.ds(ch * ECH, ECH)]], vals_v)
        pltpu.sync_copy(vals_v, g_sh.at[ridx], add=True)
    plsc.subcore_barrier()
    pltpu.sync_copy(g_sh.at[sl], out_hbm.at[cid, sl])


_edge = functools.partial(
    pl.kernel,
    out_type=jax.ShapeDtypeStruct((NC, N_PAD), jnp.float32),
    mesh=plsc.VectorSubcoreMesh(core_axis_name="c", subcore_axis_name="s", num_cores=NC, num_subcores=NS),
    scratch_types=[
        pltpu.VMEM((ECH,), jnp.int32),
        pltpu.VMEM((ECH,), jnp.int32),
        pltpu.VMEM((ECH,), jnp.int32),
        pltpu.VMEM((ECH,), jnp.int32),
        pltpu.VMEM((EPT,), jnp.int32),
        pltpu.VMEM((ECH,), jnp.float32),
        pltpu.VMEM_SHARED((N_PAD,), jnp.float32),
        pltpu.VMEM_SHARED((N_PAD,), jnp.float32),
    ],
)(_edge_body)


def _final_body(sc0_ref, dinv_ref, t_ref, g0_ref, g1_ref, c_ref, out_ref):
    out_ref[...] = sc0_ref[...] + c_ref[0] + dinv_ref[...] * (
        g0_ref[...] + g1_ref[...] + t_ref[...])


def _final(sc02d, dinv2d, t2d, g0, g1, cconst):
    return pl.pallas_call(
        _final_body,
        grid=(NB,),
        in_specs=[pl.BlockSpec((BLK, LANE), lambda i: (i, 0))] * 5
        + [pl.BlockSpec(memory_space=pltpu.SMEM)],
        out_specs=pl.BlockSpec((BLK, LANE), lambda i: (i, 0)),
        out_shape=jax.ShapeDtypeStruct((RN, LANE), jnp.float32),
    )(sc02d, dinv2d, t2d, g0, g1, cconst)


def kernel(x, edge_index, x_face, W1, b1, gamma, beta, prelu_a, W2, b2,
           w0_w, w0_b, gcn_w, gcn_b, wb_w, wb_b):
    f32 = jnp.float32
    row = edge_index[0].astype(jnp.int32)
    col = edge_index[1].astype(jnp.int32)
    pad = jnp.full((E_PAD - E,), N, jnp.int32)
    rows1d = jnp.concatenate([row, pad])
    cols1d = jnp.concatenate([col, pad])

    xp = jnp.zeros((N_PAD, 2), f32).at[:N].set(x)
    zeros_n = jnp.zeros((N_PAD,), f32)

    gw = gcn_w @ wb_w
    vmat = jnp.concatenate([W2 @ gw, W2 @ w0_w], axis=1)
    cb = (b2 @ gw)[0]
    c0 = (b2 @ w0_w)[0] + w0_b[0]
    cconst = ((gcn_b @ wb_w)[0] + wb_b[0]).reshape(1)
    scal = jnp.stack([cb, c0, prelu_a[0]])

    mom = _moments(xp)
    s2d, sc02d = _node(xp, mom, W1, gamma.reshape(1, H), beta.reshape(1, H),
                       vmat, scal)

    ones_t = jnp.ones((EPT,), f32)
    degp = _hist(rows1d, zeros_n, ones_t)
    d0 = degp[0].reshape(RN, LANE)
    d1 = degp[1].reshape(RN, LANE)
    t2d, dinv2d = _dinv_t(d0, d1, s2d)

    gp = _edge(rows1d, cols1d, t2d.reshape(N_PAD), zeros_n)
    g0 = gp[0].reshape(RN, LANE)
    g1 = gp[1].reshape(RN, LANE)

    out2d = _final(sc02d, dinv2d, t2d, g0, g1, cconst)
    return out2d.reshape(N_PAD)[:N]


# speedup vs baseline: 99.8277x; 1.6543x over previous
import functools

import jax
import jax.numpy as jnp
from jax import lax
from jax.experimental import pallas as pl
from jax.experimental.pallas import tpu as pltpu
from jax.experimental.pallas import tpu_sc as plsc

N = 100000
E = 1600000
H = 32

LANE = 128
N_PAD = 100352
RN = N_PAD // LANE
BLK = 112
NB = RN // BLK
BN = BLK * LANE

NC = 2
NS = 16
NW = NC * NS
SLICE = N_PAD // NS

EPT = E // NW
ECH = EPT // 5


def _moments_body(x_ref, out_ref, acc_ref):
    i = pl.program_id(0)

    @pl.when(i == 0)
    def _init():
        for k in range(5):
            acc_ref[k] = 0.0

    xb = x_ref[...]
    rid = lax.broadcasted_iota(jnp.int32, (BN, 2), 0) + i * BN
    xb = jnp.where(rid < N, xb, 0.0)
    c0 = xb[:, 0]
    c1 = xb[:, 1]
    acc_ref[0] += jnp.sum(c0)
    acc_ref[1] += jnp.sum(c1)
    acc_ref[2] += jnp.sum(c0 * c0)
    acc_ref[3] += jnp.sum(c1 * c1)
    acc_ref[4] += jnp.sum(c0 * c1)

    @pl.when(i == NB - 1)
    def _fin():
        for k in range(5):
            out_ref[k] = acc_ref[k]


def _moments(xp):
    return pl.pallas_call(
        _moments_body,
        grid=(NB,),
        in_specs=[pl.BlockSpec((BN, 2), lambda i: (i, 0))],
        out_specs=pl.BlockSpec(memory_space=pltpu.SMEM),
        out_shape=jax.ShapeDtypeStruct((8,), jnp.float32),
        scratch_shapes=[pltpu.SMEM((8,), jnp.float32)],
    )(xp)


def _node_body(x_ref, mom_ref, w1_ref, gam_ref, bet_ref, vmat_ref, sc_ref,
               s_ref, sc0_ref):
    i = pl.program_id(0)
    inv_n = 1.0 / N
    m0 = mom_ref[0] * inv_n
    m1 = mom_ref[1] * inv_n
    v00 = mom_ref[2] * inv_n - m0 * m0
    v11 = mom_ref[3] * inv_n - m1 * m1
    v01 = mom_ref[4] * inv_n - m0 * m1

    w1 = w1_ref[...]
    w1a = w1[0:1, :]
    w1b = w1[1:2, :]
    varh = v00 * w1a * w1a + 2.0 * v01 * w1a * w1b + v11 * w1b * w1b
    a = gam_ref[...] * lax.rsqrt(varh + 1e-5)
    muc = m0 * w1a + m1 * w1b

    xb = x_ref[...]
    hc = xb[:, 0:1] * w1a + xb[:, 1:2] * w1b - muc
    hn = hc * a + bet_ref[...]
    pa = sc_ref[2]
    act = jnp.where(hn > 0, hn, pa * hn)

    out2 = jnp.dot(act, vmat_ref[...], preferred_element_type=jnp.float32,
                   precision=lax.Precision.HIGHEST)
    sb = out2[:, 0] + sc_ref[0]
    sc0b = out2[:, 1] + sc_ref[1]
    ids = lax.broadcasted_iota(jnp.int32, (BN,), 0) + i * BN
    sb = jnp.where(ids < N, sb, 0.0)
    s_ref[...] = sb.reshape(BLK, LANE)
    sc0_ref[...] = sc0b.reshape(BLK, LANE)


def _node(xp, mom, w1, gam, bet, vmat, scal):
    return pl.pallas_call(
        _node_body,
        grid=(NB,),
        in_specs=[
            pl.BlockSpec((BN, 2), lambda i: (i, 0)),
            pl.BlockSpec(memory_space=pltpu.SMEM),
            pl.BlockSpec((2, H), lambda i: (0, 0)),
            pl.BlockSpec((1, H), lambda i: (0, 0)),
            pl.BlockSpec((1, H), lambda i: (0, 0)),
            pl.BlockSpec((H, 2), lambda i: (0, 0)),
            pl.BlockSpec(memory_space=pltpu.SMEM),
        ],
        out_specs=[
            pl.BlockSpec((BLK, LANE), lambda i: (i, 0)),
            pl.BlockSpec((BLK, LANE), lambda i: (i, 0)),
        ],
        out_shape=[
            jax.ShapeDtypeStruct((RN, LANE), jnp.float32),
            jax.ShapeDtypeStruct((RN, LANE), jnp.float32),
        ],
    )(xp, mom, w1, gam, bet, vmat, scal)


def _hist_body(ei_hbm, zeros_hbm, ones_hbm, out_hbm, idx_v, ones_v, deg_sh):
    cid = lax.axis_index("c")
    sid = lax.axis_index("s")
    wid = cid * NS + sid
    sl = pl.ds(sid * SLICE, SLICE)
    pltpu.sync_copy(zeros_hbm.at[sl], deg_sh.at[sl])
    pltpu.sync_copy(ones_hbm, ones_v)
    pltpu.sync_copy(ei_hbm.at[pl.ds(wid * EPT, EPT)], idx_v)
    plsc.subcore_barrier()
    pltpu.sync_copy(ones_v, deg_sh.at[idx_v], add=True)
    plsc.subcore_barrier()
    pltpu.sync_copy(deg_sh.at[sl], out_hbm.at[cid, sl])


_hist = functools.partial(
    pl.kernel,
    out_type=jax.ShapeDtypeStruct((NC, N_PAD), jnp.float32),
    mesh=plsc.VectorSubcoreMesh(core_axis_name="c", subcore_axis_name="s", num_cores=NC, num_subcores=NS),
    scratch_types=[
        pltpu.VMEM((EPT,), jnp.int32),
        pltpu.VMEM((EPT,), jnp.float32),
        pltpu.VMEM_SHARED((N_PAD,), jnp.float32),
    ],
)(_hist_body)


def _dinv_body(d0_ref, d1_ref, s_ref, t_ref, dinv_ref):
    deg = d0_ref[...] + d1_ref[...] + 1.0
    dinv = lax.rsqrt(deg)
    dinv_ref[...] = dinv
    t_ref[...] = dinv * s_ref[...]


def _dinv_t(d0, d1, s2d):
    return pl.pallas_call(
        _dinv_body,
        grid=(NB,),
        in_specs=[pl.BlockSpec((BLK, LANE), lambda i: (i, 0))] * 3,
        out_specs=[pl.BlockSpec((BLK, LANE), lambda i: (i, 0))] * 2,
        out_shape=[
            jax.ShapeDtypeStruct((RN, LANE), jnp.float32),
            jax.ShapeDtypeStruct((RN, LANE), jnp.float32),
        ],
    )(d0, d1, s2d)


def _edge_body(ei_hbm, t_hbm, zeros_hbm, out_hbm,
               ridx_0, ridx_1, ridx_2, ridx_3, ridx_4, cidx_v, vals_v,
               t_sh, g_sh):
    ridx_bufs = (ridx_0, ridx_1, ridx_2, ridx_3, ridx_4)
    cid = lax.axis_index("c")
    sid = lax.axis_index("s")
    wid = cid * NS + sid
    sl = pl.ds(sid * SLICE, SLICE)
    pltpu.sync_copy(t_hbm.at[sl], t_sh.at[sl])
    pltpu.sync_copy(zeros_hbm.at[sl], g_sh.at[sl])
    base = wid * EPT
    pltpu.sync_copy(ei_hbm.at[pl.ds(E + base, EPT)], cidx_v)
    for ch, ridx in enumerate(ridx_bufs):
        pltpu.sync_copy(ei_hbm.at[pl.ds(base + ch * ECH, ECH)], ridx)
    plsc.subcore_barrier()
    for ch, ridx in enumerate(ridx_bufs):
        pltpu.sync_copy(t_sh.at[cidx_v.at[pl.ds(ch * ECH, ECH)]], vals_v)
        pltpu.sync_copy(vals_v, g_sh.at[ridx], add=True)
    plsc.subcore_barrier()
    pltpu.sync_copy(g_sh.at[sl], out_hbm.at[cid, sl])


_edge = functools.partial(
    pl.kernel,
    out_type=jax.ShapeDtypeStruct((NC, N_PAD), jnp.float32),
    mesh=plsc.VectorSubcoreMesh(core_axis_name="c", subcore_axis_name="s", num_cores=NC, num_subcores=NS),
    scratch_types=[
        pltpu.VMEM((ECH,), jnp.int32),
        pltpu.VMEM((ECH,), jnp.int32),
        pltpu.VMEM((ECH,), jnp.int32),
        pltpu.VMEM((ECH,), jnp.int32),
        pltpu.VMEM((ECH,), jnp.int32),
        pltpu.VMEM((EPT,), jnp.int32),
        pltpu.VMEM((ECH,), jnp.float32),
        pltpu.VMEM_SHARED((N_PAD,), jnp.float32),
        pltpu.VMEM_SHARED((N_PAD,), jnp.float32),
    ],
)(_edge_body)


def _final_body(sc0_ref, dinv_ref, t_ref, g0_ref, g1_ref, c_ref, out_ref):
    out_ref[...] = sc0_ref[...] + c_ref[0] + dinv_ref[...] * (
        g0_ref[...] + g1_ref[...] + t_ref[...])


def _final(sc02d, dinv2d, t2d, g0, g1, cconst):
    return pl.pallas_call(
        _final_body,
        grid=(NB,),
        in_specs=[pl.BlockSpec((BLK, LANE), lambda i: (i, 0))] * 5
        + [pl.BlockSpec(memory_space=pltpu.SMEM)],
        out_specs=pl.BlockSpec((BLK, LANE), lambda i: (i, 0)),
        out_shape=jax.ShapeDtypeStruct((RN, LANE), jnp.float32),
    )(sc02d, dinv2d, t2d, g0, g1, cconst)


def kernel(x, edge_index, x_face, W1, b1, gamma, beta, prelu_a, W2, b2,
           w0_w, w0_b, gcn_w, gcn_b, wb_w, wb_b):
    f32 = jnp.float32
    ei = edge_index.astype(jnp.int32).reshape(2 * E)

    xp = jnp.zeros((N_PAD, 2), f32).at[:N].set(x)
    zeros_n = jnp.zeros((N_PAD,), f32)

    gw = gcn_w @ wb_w
    vmat = jnp.concatenate([W2 @ gw, W2 @ w0_w], axis=1)
    cb = (b2 @ gw)[0]
    c0 = (b2 @ w0_w)[0] + w0_b[0]
    cconst = ((gcn_b @ wb_w)[0] + wb_b[0]).reshape(1)
    scal = jnp.stack([cb, c0, prelu_a[0]])

    mom = _moments(xp)
    s2d, sc02d = _node(xp, mom, W1, gamma.reshape(1, H), beta.reshape(1, H),
                       vmat, scal)

    ones_t = jnp.ones((EPT,), f32)
    degp = _hist(ei, zeros_n, ones_t)
    d0 = degp[0].reshape(RN, LANE)
    d1 = degp[1].reshape(RN, LANE)
    t2d, dinv2d = _dinv_t(d0, d1, s2d)

    gp = _edge(ei, t2d.reshape(N_PAD), zeros_n)
    g0 = gp[0].reshape(RN, LANE)
    g1 = gp[1].reshape(RN, LANE)

    out2d = _final(sc02d, dinv2d, t2d, g0, g1, cconst)
    return out2d.reshape(N_PAD)[:N]


# speedup vs baseline: 267.2480x; 2.6771x over previous
import functools

import jax
import jax.numpy as jnp
from jax import lax
from jax.experimental import pallas as pl
from jax.experimental.pallas import tpu as pltpu
from jax.experimental.pallas import tpu_sc as plsc

N = 100000
E = 1600000
H = 32

LANE = 128
N_PAD = 100352
RN = N_PAD // LANE
BLK = 112
NB = RN // BLK
BN = BLK * LANE

NC = 2
NS = 16
NW = NC * NS
SLICE = N_PAD // NS

EPT = E // NW
ECH = EPT // 5


def _moments_body(x0_ref, x1_ref, out_ref, acc_ref):
    i = pl.program_id(0)

    @pl.when(i == 0)
    def _init():
        for k in range(5):
            acc_ref[k] = 0.0

    c0 = x0_ref[0]
    c1 = x1_ref[0]
    acc_ref[0] += jnp.sum(c0)
    acc_ref[1] += jnp.sum(c1)
    acc_ref[2] += jnp.sum(c0 * c0)
    acc_ref[3] += jnp.sum(c1 * c1)
    acc_ref[4] += jnp.sum(c0 * c1)

    @pl.when(i == NB - 1)
    def _fin():
        for k in range(5):
            out_ref[k] = acc_ref[k]


def _moments(x0c, x1c):
    return pl.pallas_call(
        _moments_body,
        grid=(NB,),
        in_specs=[pl.BlockSpec((1, BLK, LANE), lambda i: (i, 0, 0))] * 2,
        out_specs=pl.BlockSpec(memory_space=pltpu.SMEM),
        out_shape=jax.ShapeDtypeStruct((8,), jnp.float32),
        scratch_shapes=[pltpu.SMEM((8,), jnp.float32)],
    )(x0c, x1c)


def _node_body(x0_ref, x1_ref, mom_ref, w1_ref, gam_ref, bet_ref,
               vb_ref, v0_ref, sc_ref, s_ref, sc0_ref):
    i = pl.program_id(0)
    inv_n = 1.0 / N
    m0 = mom_ref[0] * inv_n
    m1 = mom_ref[1] * inv_n
    v00 = mom_ref[2] * inv_n - m0 * m0
    v11 = mom_ref[3] * inv_n - m1 * m1
    v01 = mom_ref[4] * inv_n - m0 * m1
    pa = sc_ref[2]

    c0 = x0_ref[0]
    c1 = x1_ref[0]
    s_acc = jnp.zeros((BLK, LANE), jnp.float32)
    sc0_acc = jnp.zeros((BLK, LANE), jnp.float32)
    for j in range(H):
        wa = w1_ref[0, j]
        wb = w1_ref[1, j]
        varh = v00 * wa * wa + 2.0 * v01 * wa * wb + v11 * wb * wb
        aj = gam_ref[j] * lax.rsqrt(varh + 1e-5)
        bj = bet_ref[j] - (m0 * wa + m1 * wb) * aj
        hn = (c0 * wa + c1 * wb) * aj + bj
        act = jnp.where(hn > 0, hn, pa * hn)
        s_acc = s_acc + act * vb_ref[j]
        sc0_acc = sc0_acc + act * v0_ref[j]

    ids = (lax.broadcasted_iota(jnp.int32, (BLK, LANE), 0) * LANE
           + lax.broadcasted_iota(jnp.int32, (BLK, LANE), 1) + i * BN)
    s_ref[0] = jnp.where(ids < N, s_acc + sc_ref[0], 0.0)
    sc0_ref[0] = sc0_acc + sc_ref[1]


def _node(x0c, x1c, mom, w1, gam, bet, vb, v0, scal):
    return pl.pallas_call(
        _node_body,
        grid=(NB,),
        in_specs=[
            pl.BlockSpec((1, BLK, LANE), lambda i: (i, 0, 0)),
            pl.BlockSpec((1, BLK, LANE), lambda i: (i, 0, 0)),
            pl.BlockSpec(memory_space=pltpu.SMEM),
            pl.BlockSpec(memory_space=pltpu.SMEM),
            pl.BlockSpec(memory_space=pltpu.SMEM),
            pl.BlockSpec(memory_space=pltpu.SMEM),
            pl.BlockSpec(memory_space=pltpu.SMEM),
            pl.BlockSpec(memory_space=pltpu.SMEM),
            pl.BlockSpec(memory_space=pltpu.SMEM),
        ],
        out_specs=[
            pl.BlockSpec((1, BLK, LANE), lambda i: (i, 0, 0)),
            pl.BlockSpec((1, BLK, LANE), lambda i: (i, 0, 0)),
        ],
        out_shape=[
            jax.ShapeDtypeStruct((NB, BLK, LANE), jnp.float32),
            jax.ShapeDtypeStruct((NB, BLK, LANE), jnp.float32),
        ],
    )(x0c, x1c, mom, w1, gam, bet, vb, v0, scal)


def _hist_body(ei_hbm, zeros_hbm, ones_hbm, out_hbm, idx_v, ones_v, deg_sh):
    cid = lax.axis_index("c")
    sid = lax.axis_index("s")
    wid = cid * NS + sid
    sl = pl.ds(sid * SLICE, SLICE)
    pltpu.sync_copy(zeros_hbm.at[sl], deg_sh.at[sl])
    pltpu.sync_copy(ones_hbm, ones_v)
    pltpu.sync_copy(ei_hbm.at[pl.ds(wid * EPT, EPT)], idx_v)
    plsc.subcore_barrier()
    pltpu.sync_copy(ones_v, deg_sh.at[idx_v], add=True)
    plsc.subcore_barrier()
    pltpu.sync_copy(deg_sh.at[sl], out_hbm.at[cid, sl])


_hist = functools.partial(
    pl.kernel,
    out_type=jax.ShapeDtypeStruct((NC, N_PAD), jnp.float32),
    mesh=plsc.VectorSubcoreMesh(core_axis_name="c", subcore_axis_name="s", num_cores=NC, num_subcores=NS),
    scratch_types=[
        pltpu.VMEM((EPT,), jnp.int32),
        pltpu.VMEM((EPT,), jnp.float32),
        pltpu.VMEM_SHARED((N_PAD,), jnp.float32),
    ],
)(_hist_body)


def _dinv_body(d0_ref, d1_ref, s_ref, t_ref, dinv_ref):
    deg = d0_ref[...] + d1_ref[...] + 1.0
    dinv = lax.rsqrt(deg)
    dinv_ref[...] = dinv
    t_ref[...] = dinv * s_ref[...]


def _dinv_t(d0, d1, s2d):
    return pl.pallas_call(
        _dinv_body,
        grid=(NB,),
        in_specs=[pl.BlockSpec((BLK, LANE), lambda i: (i, 0))] * 3,
        out_specs=[pl.BlockSpec((BLK, LANE), lambda i: (i, 0))] * 2,
        out_shape=[
            jax.ShapeDtypeStruct((RN, LANE), jnp.float32),
            jax.ShapeDtypeStruct((RN, LANE), jnp.float32),
        ],
    )(d0, d1, s2d)


def _edge_body(ei_hbm, t_hbm, zeros_hbm, out_hbm,
               ridx_0, ridx_1, ridx_2, ridx_3, ridx_4, cidx_v, vals_v,
               t_sh, g_sh):
    ridx_bufs = (ridx_0, ridx_1, ridx_2, ridx_3, ridx_4)
    cid = lax.axis_index("c")
    sid = lax.axis_index("s")
    wid = cid * NS + sid
    sl = pl.ds(sid * SLICE, SLICE)
    pltpu.sync_copy(t_hbm.at[sl], t_sh.at[sl])
    pltpu.sync_copy(zeros_hbm.at[sl], g_sh.at[sl])
    base = wid * EPT
    pltpu.sync_copy(ei_hbm.at[pl.ds(E + base, EPT)], cidx_v)
    for ch, ridx in enumerate(ridx_bufs):
        pltpu.sync_copy(ei_hbm.at[pl.ds(base + ch * ECH, ECH)], ridx)
    plsc.subcore_barrier()
    for ch, ridx in enumerate(ridx_bufs):
        pltpu.sync_copy(t_sh.at[cidx_v.at[pl.ds(ch * ECH, ECH)]], vals_v)
        pltpu.sync_copy(vals_v, g_sh.at[ridx], add=True)
    plsc.subcore_barrier()
    pltpu.sync_copy(g_sh.at[sl], out_hbm.at[cid, sl])


_edge = functools.partial(
    pl.kernel,
    out_type=jax.ShapeDtypeStruct((NC, N_PAD), jnp.float32),
    mesh=plsc.VectorSubcoreMesh(core_axis_name="c", subcore_axis_name="s", num_cores=NC, num_subcores=NS),
    scratch_types=[
        pltpu.VMEM((ECH,), jnp.int32),
        pltpu.VMEM((ECH,), jnp.int32),
        pltpu.VMEM((ECH,), jnp.int32),
        pltpu.VMEM((ECH,), jnp.int32),
        pltpu.VMEM((ECH,), jnp.int32),
        pltpu.VMEM((EPT,), jnp.int32),
        pltpu.VMEM((ECH,), jnp.float32),
        pltpu.VMEM_SHARED((N_PAD,), jnp.float32),
        pltpu.VMEM_SHARED((N_PAD,), jnp.float32),
    ],
)(_edge_body)


def _final_body(sc0_ref, dinv_ref, t_ref, g0_ref, g1_ref, c_ref, out_ref):
    out_ref[...] = sc0_ref[...] + c_ref[0] + dinv_ref[...] * (
        g0_ref[...] + g1_ref[...] + t_ref[...])


def _final(sc02d, dinv2d, t2d, g0, g1, cconst):
    return pl.pallas_call(
        _final_body,
        grid=(NB,),
        in_specs=[pl.BlockSpec((BLK, LANE), lambda i: (i, 0))] * 5
        + [pl.BlockSpec(memory_space=pltpu.SMEM)],
        out_specs=pl.BlockSpec((BLK, LANE), lambda i: (i, 0)),
        out_shape=jax.ShapeDtypeStruct((RN, LANE), jnp.float32),
    )(sc02d, dinv2d, t2d, g0, g1, cconst)


def kernel(x, edge_index, x_face, W1, b1, gamma, beta, prelu_a, W2, b2,
           w0_w, w0_b, gcn_w, gcn_b, wb_w, wb_b):
    f32 = jnp.float32
    ei = edge_index.astype(jnp.int32).reshape(2 * E)

    x0c = jnp.pad(x[:, 0], (0, N_PAD - N)).reshape(NB, BLK, LANE)
    x1c = jnp.pad(x[:, 1], (0, N_PAD - N)).reshape(NB, BLK, LANE)
    zeros_n = jnp.zeros((N_PAD,), f32)

    gw = gcn_w @ wb_w
    vmat = jnp.concatenate([W2 @ gw, W2 @ w0_w], axis=1)
    cb = (b2 @ gw)[0]
    c0 = (b2 @ w0_w)[0] + w0_b[0]
    cconst = ((gcn_b @ wb_w)[0] + wb_b[0]).reshape(1)
    scal = jnp.stack([cb, c0, prelu_a[0]])

    mom = _moments(x0c, x1c)
    s3d, sc03d = _node(x0c, x1c, mom, W1, gamma, beta,
                       vmat[:, 0], vmat[:, 1], scal)
    s2d = s3d.reshape(RN, LANE)
    sc02d = sc03d.reshape(RN, LANE)

    ones_t = jnp.ones((EPT,), f32)
    degp = _hist(ei, zeros_n, ones_t)
    d0 = degp[0].reshape(RN, LANE)
    d1 = degp[1].reshape(RN, LANE)
    t2d, dinv2d = _dinv_t(d0, d1, s2d)

    gp = _edge(ei, t2d.reshape(N_PAD), zeros_n)
    g0 = gp[0].reshape(RN, LANE)
    g1 = gp[1].reshape(RN, LANE)

    out2d = _final(sc02d, dinv2d, t2d, g0, g1, cconst)
    return out2d.reshape(N_PAD)[:N]
